# baseline (speedup 1.0000x reference)
import functools

import jax
import jax.numpy as jnp
from jax import lax
from jax.experimental import pallas as pl
from jax.experimental.pallas import tpu as pltpu
from jax.experimental.pallas import tpu_sc as plsc

B = 16384
D = 128
C = 1000
CPAD = 1024
NC = 2
NS = 16
NW = NC * NS
RPW = B // NW
SUB = 128
NSUB = RPW // SUB
NBUF = 3
STRIPE = CPAD // NS


def _sc_body(pred_hbm, y_hbm, zrow_hbm,
             st_out, n_out,
             idx0, idx1, idx2, lbl0, lbl1, lbl2, p0, p1, p2, q0, q1,
             hist_sm, nv_v,
             acc_s, acc_t,
             sem_in0, sem_in1, sem_in2, sem_sc0, sem_sc1, sem_sc2):
    cid = lax.axis_index("c")
    sid = lax.axis_index("s")
    wid = sid * NC + cid
    idx_bufs = (idx0, idx1, idx2)
    lbl_bufs = (lbl0, lbl1, lbl2)
    lane = lax.iota(jnp.int32, 16)
    p_bufs = (p0, p1, p2)
    q_bufs = (q0, q1)
    sem_in = (sem_in0, sem_in1, sem_in2)
    sem_sc = (sem_sc0, sem_sc1, sem_sc2)

    def start_fill(s):
        buf = s % NBUF
        base = wid * RPW + s * SUB
        pltpu.sync_copy(y_hbm.at[pl.ds(base, SUB)], idx_bufs[buf])
        pltpu.sync_copy(y_hbm.at[pl.ds(base, SUB)], lbl_bufs[buf])
        return pltpu.async_copy(
            pred_hbm.at[pl.ds(base, SUB)], p_bufs[buf], sem_in[buf])

    fills = {0: start_fill(0)}

    rz = pl.ds(sid * STRIPE, STRIPE)
    pltpu.sync_copy(zrow_hbm.at[rz], acc_s.at[rz])
    pltpu.sync_copy(zrow_hbm.at[rz], acc_t.at[rz])

    def zh_body(i, carry):
        hist_sm[i] = 0
        return carry

    lax.fori_loop(0, CPAD, zh_body, 0)
    plsc.subcore_barrier()

    scats = {}
    for s in range(NSUB):
        buf = s % NBUF
        if s - 2 in scats:
            for d in scats[s - 2]:
                d.wait()
        fills[s].wait()
        pv = p_bufs[buf]
        qv = q_bufs[s % 2]

        @plsc.parallel_loop(0, SUB, unroll=4)
        def sq_body(r, pv=pv, qv=qv):
            for c in range(D // 16):
                v = pv[r, pl.ds(c * 16, 16)]
                qv[r, pl.ds(c * 16, 16)] = v * v

        if s + 1 < NSUB:
            fills[s + 1] = start_fill(s + 1)

        scats[s] = (
            pltpu.async_copy(pv, acc_s.at[idx_bufs[buf]], sem_sc[buf],
                             add=True),
            pltpu.async_copy(qv, acc_t.at[idx_bufs[buf]], sem_sc[buf],
                             add=True),
        )

        lbl_cur = lbl_bufs[buf]

        def h_body(g, carry, lbl_cur=lbl_cur):
            lblv = lbl_cur[pl.ds(g * 16, 16)]
            for k in range(16):
                lbl = lblv[k]
                hist_sm[lbl] = hist_sm[lbl] + 1
            return carry

        lax.fori_loop(0, SUB // 16, h_body, 0)

    for s in (NSUB - 2, NSUB - 1):
        if s in scats:
            for d in scats[s]:
                d.wait()
    def cv_body(g, carry):
        v = jnp.zeros((16,), jnp.int32)
        for k in range(16):
            v = jnp.where(lane == k,
                          jnp.full((16,), hist_sm[g * 16 + k], jnp.int32), v)
        nv_v[pl.ds(g * 16, 16)] = v
        return carry

    lax.fori_loop(0, CPAD // 16, cv_body, 0)
    plsc.subcore_barrier()
    pltpu.sync_copy(acc_s.at[rz], st_out.at[0, cid, rz])
    pltpu.sync_copy(acc_t.at[rz], st_out.at[1, cid, rz])
    pltpu.sync_copy(nv_v, n_out.at[wid])


_sc_segsums = functools.partial(
    pl.kernel,
    mesh=plsc.VectorSubcoreMesh(core_axis_name="c", subcore_axis_name="s"),
    out_type=[
        jax.ShapeDtypeStruct((2, NC, CPAD, D), jnp.float32),
        jax.ShapeDtypeStruct((NW, CPAD), jnp.int32),
    ],
    scratch_types=[
        pltpu.VMEM((SUB,), jnp.int32),
        pltpu.VMEM((SUB,), jnp.int32),
        pltpu.VMEM((SUB,), jnp.int32),
        pltpu.VMEM((SUB,), jnp.int32),
        pltpu.VMEM((SUB,), jnp.int32),
        pltpu.VMEM((SUB,), jnp.int32),
        pltpu.VMEM((SUB, D), jnp.float32),
        pltpu.VMEM((SUB, D), jnp.float32),
        pltpu.VMEM((SUB, D), jnp.float32),
        pltpu.VMEM((SUB, D), jnp.float32),
        pltpu.VMEM((SUB, D), jnp.float32),
        pltpu.SMEM((CPAD,), jnp.int32),
        pltpu.VMEM((CPAD,), jnp.int32),
        pltpu.VMEM_SHARED((CPAD, D), jnp.float32),
        pltpu.VMEM_SHARED((CPAD, D), jnp.float32),
        pltpu.SemaphoreType.DMA,
        pltpu.SemaphoreType.DMA,
        pltpu.SemaphoreType.DMA,
        pltpu.SemaphoreType.DMA,
        pltpu.SemaphoreType.DMA,
        pltpu.SemaphoreType.DMA,
    ],
)(_sc_body)


def _tc_finish(st_ref, n_ref, c_ref, o_ref):
    s2 = (st_ref[0, 0] + st_ref[0, 1])[:C]
    t2 = (st_ref[1, 0] + st_ref[1, 1])[:C]
    q = jnp.sum(t2, axis=1, keepdims=True)
    n = jnp.sum(n_ref[...], axis=0)[:C].astype(jnp.float32).reshape(C, 1)
    ctr = c_ref[...]
    cross = jnp.sum(ctr * s2, axis=1, keepdims=True)
    cc = jnp.sum(ctr * ctr, axis=1, keepdims=True)
    means = jnp.where(n > 0.0,
                      (q - 2.0 * cross) / jnp.maximum(n, 1.0) + cc,
                      0.0)
    o_ref[...] = jnp.sum(means).reshape(1, 1)


def kernel(pred, target_y, centers):
    y = target_y.astype(jnp.int32)
    zrow = jnp.zeros((CPAD, D), jnp.float32)
    st_p, n_p = _sc_segsums(pred, y, zrow)
    loss = pl.pallas_call(
        _tc_finish,
        out_shape=jax.ShapeDtypeStruct((1, 1), jnp.float32),
    )(st_p, n_p, centers)
    return loss.reshape(1)


# speedup vs baseline: 1.1459x; 1.1459x over previous
import functools

import jax
import jax.numpy as jnp
from jax import lax
from jax.experimental import pallas as pl
from jax.experimental.pallas import tpu as pltpu
from jax.experimental.pallas import tpu_sc as plsc

B = 16384
D = 128
C = 1000
CPAD = 1024
NC = 2
NS = 16
NW = NC * NS
RPW = B // NW
SUB = 128
NSUB = RPW // SUB
NBUF = 3
STRIPE = CPAD // NS


def _sc_body(pred_hbm, y_hbm, zrow_hbm,
             st_out, n_out,
             idx0, idx1, idx2, lbl0, lbl1, lbl2, p0, p1, p2, q0, q1,
             hist_sm, nv_v,
             acc_s, acc_t,
             sem_in0, sem_in1, sem_in2, sem_sc0, sem_sc1, sem_sc2):
    cid = lax.axis_index("c")
    sid = lax.axis_index("s")
    wid = sid * NC + cid
    idx_bufs = (idx0, idx1, idx2)
    lbl_bufs = (lbl0, lbl1, lbl2)
    lane = lax.iota(jnp.int32, 16)
    p_bufs = (p0, p1, p2)
    q_bufs = (q0, q1)
    sem_in = (sem_in0, sem_in1, sem_in2)
    sem_sc = (sem_sc0, sem_sc1, sem_sc2)

    def start_fill(s):
        buf = s % NBUF
        base = wid * RPW + s * SUB
        pltpu.sync_copy(y_hbm.at[pl.ds(base, SUB)], idx_bufs[buf])
        pltpu.sync_copy(y_hbm.at[pl.ds(base, SUB)], lbl_bufs[buf])
        return pltpu.async_copy(
            pred_hbm.at[pl.ds(base, SUB)], p_bufs[buf], sem_in[buf])

    fills = {0: start_fill(0)}

    rz = pl.ds(sid * STRIPE, STRIPE)
    z1 = pltpu.async_copy(zrow_hbm.at[rz], acc_s.at[rz], sem_sc0)
    z2 = pltpu.async_copy(zrow_hbm.at[rz], acc_t.at[rz], sem_sc1)

    def zh_body(i, carry):
        for k in range(16):
            hist_sm[i * 16 + k] = 0
        return carry

    lax.fori_loop(0, 63, zh_body, 0)
    z1.wait()
    z2.wait()
    plsc.subcore_barrier()

    scats = {}
    for s in range(NSUB):
        buf = s % NBUF
        if s - 2 in scats:
            for d in scats[s - 2]:
                d.wait()
        fills[s].wait()
        pv = p_bufs[buf]
        qv = q_bufs[s % 2]

        @plsc.parallel_loop(0, SUB, unroll=4)
        def sq_body(r, pv=pv, qv=qv):
            for c in range(D // 16):
                v = pv[r, pl.ds(c * 16, 16)]
                qv[r, pl.ds(c * 16, 16)] = v * v

        if s + 1 < NSUB:
            fills[s + 1] = start_fill(s + 1)

        scats[s] = (
            pltpu.async_copy(pv, acc_s.at[idx_bufs[buf]], sem_sc[buf],
                             add=True),
            pltpu.async_copy(qv, acc_t.at[idx_bufs[buf]], sem_sc[buf],
                             add=True),
        )

        lbl_cur = lbl_bufs[buf]

        def h_body(g, carry, lbl_cur=lbl_cur):
            lblv = lbl_cur[pl.ds(g * 16, 16)]
            for k in range(16):
                lbl = lblv[k]
                hist_sm[lbl] = hist_sm[lbl] + 1
            return carry

        lax.fori_loop(0, SUB // 16, h_body, 0)

    def cv_body(g, carry):
        v = jnp.zeros((16,), jnp.int32)
        for k in range(16):
            v = jnp.where(lane == k,
                          jnp.full((16,), hist_sm[g * 16 + k], jnp.int32), v)
        nv_v[pl.ds(g * 16, 16)] = v
        return carry

    lax.fori_loop(0, 63, cv_body, 0)
    for s in (NSUB - 2, NSUB - 1):
        if s in scats:
            for d in scats[s]:
                d.wait()
    plsc.subcore_barrier()
    pltpu.sync_copy(acc_s.at[rz], st_out.at[0, cid, rz])
    pltpu.sync_copy(acc_t.at[rz], st_out.at[1, cid, rz])
    pltpu.sync_copy(nv_v, n_out.at[wid])


_sc_segsums = functools.partial(
    pl.kernel,
    mesh=plsc.VectorSubcoreMesh(core_axis_name="c", subcore_axis_name="s"),
    out_type=[
        jax.ShapeDtypeStruct((2, NC, CPAD, D), jnp.float32),
        jax.ShapeDtypeStruct((NW, CPAD), jnp.int32),
    ],
    scratch_types=[
        pltpu.VMEM((SUB,), jnp.int32),
        pltpu.VMEM((SUB,), jnp.int32),
        pltpu.VMEM((SUB,), jnp.int32),
        pltpu.VMEM((SUB,), jnp.int32),
        pltpu.VMEM((SUB,), jnp.int32),
        pltpu.VMEM((SUB,), jnp.int32),
        pltpu.VMEM((SUB, D), jnp.float32),
        pltpu.VMEM((SUB, D), jnp.float32),
        pltpu.VMEM((SUB, D), jnp.float32),
        pltpu.VMEM((SUB, D), jnp.float32),
        pltpu.VMEM((SUB, D), jnp.float32),
        pltpu.SMEM((CPAD,), jnp.int32),
        pltpu.VMEM((CPAD,), jnp.int32),
        pltpu.VMEM_SHARED((CPAD, D), jnp.float32),
        pltpu.VMEM_SHARED((CPAD, D), jnp.float32),
        pltpu.SemaphoreType.DMA,
        pltpu.SemaphoreType.DMA,
        pltpu.SemaphoreType.DMA,
        pltpu.SemaphoreType.DMA,
        pltpu.SemaphoreType.DMA,
        pltpu.SemaphoreType.DMA,
    ],
)(_sc_body)


def _tc_finish(st_ref, n_ref, c_ref, o_ref):
    s2 = (st_ref[0, 0] + st_ref[0, 1])[:C]
    t2 = (st_ref[1, 0] + st_ref[1, 1])[:C]
    q = jnp.sum(t2, axis=1, keepdims=True)
    n = jnp.sum(n_ref[...], axis=0)[:C].astype(jnp.float32).reshape(C, 1)
    ctr = c_ref[...]
    cross = jnp.sum(ctr * s2, axis=1, keepdims=True)
    cc = jnp.sum(ctr * ctr, axis=1, keepdims=True)
    means = jnp.where(n > 0.0,
                      (q - 2.0 * cross) / jnp.maximum(n, 1.0) + cc,
                      0.0)
    o_ref[...] = jnp.sum(means).reshape(1, 1)


def kernel(pred, target_y, centers):
    y = target_y.astype(jnp.int32)
    zrow = jnp.zeros((CPAD, D), jnp.float32)
    st_p, n_p = _sc_segsums(pred, y, zrow)
    loss = pl.pallas_call(
        _tc_finish,
        out_shape=jax.ShapeDtypeStruct((1, 1), jnp.float32),
    )(st_p, n_p, centers)
    return loss.reshape(1)
